# baseline (speedup 1.0000x reference)
import functools

import jax
import jax.numpy as jnp
import numpy as np
from jax import lax
from jax.experimental import pallas as pl
from jax.experimental.pallas import tpu as pltpu
from jax.experimental.pallas import tpu_sc as plsc

N = 10000
E = 160000
LATENT = 256
T = 1000
HALF = 128

NC = 2
NS = 16

EB = 40
CKB = 25
NCK = 10
NBUF = 6
NCH_E = N // EB
KMAX_E = (NCH_E + NS - 1) // NS

DB = 40
DCKB = 25
DNCK = 5
NCH_D = N // DB
KMAX_D = (NCH_D + NS - 1) // NS

_mesh = plsc.VectorSubcoreMesh(core_axis_name="c", subcore_axis_name="s")


@functools.partial(
    pl.kernel,
    out_type=jax.ShapeDtypeStruct((NC, N, 16), jnp.float32),
    mesh=_mesh,
    scratch_types=[
        pltpu.VMEM_SHARED((N, 16), jnp.float32),
        pltpu.VMEM((DCKB, DB), jnp.int32),
        pltpu.VMEM((DB, 16), jnp.float32),
        pltpu.VMEM((DB, 16), jnp.float32),
        pltpu.SemaphoreType.DMA,
    ],
)
def _deg_kernel(dst_hbm, seed_hbm, ones_hbm, out_hbm, hist, ids, ones, tmp,
                dsem):
    c = lax.axis_index("c")
    s = lax.axis_index("s")
    wid = c * NS + s

    pltpu.sync_copy(ones_hbm, ones)

    def init_chunk(k, carry):
        idx = s + NS * k

        @pl.when(idx < NCH_D)
        def _():
            r0 = idx * DB
            pltpu.sync_copy(seed_hbm.at[c].at[pl.ds(r0, DB)], tmp)
            pltpu.sync_copy(tmp, hist.at[pl.ds(r0, DB)])

        return carry

    lax.fori_loop(0, KMAX_D, init_chunk, 0)
    plsc.subcore_barrier()

    def id_chunk(j, carry):
        pltpu.sync_copy(dst_hbm.at[wid].at[j], ids)

        def fire(i, c2):
            pltpu.async_copy(ones, hist.at[ids.at[i]], dsem, add=True)
            return c2

        lax.fori_loop(0, DCKB, fire, 0)

        def drain(i, c2):
            pltpu.make_async_copy(ones, hist.at[ids.at[0]], dsem).wait()
            return c2

        lax.fori_loop(0, DCKB, drain, 0)
        return carry

    lax.fori_loop(0, DNCK, id_chunk, 0)
    plsc.subcore_barrier()

    def write_chunk(k, carry):
        idx = s + NS * k

        @pl.when(idx < NCH_D)
        def _():
            r0 = idx * DB
            pltpu.sync_copy(hist.at[pl.ds(r0, DB)], tmp)
            pltpu.sync_copy(tmp, out_hbm.at[c].at[pl.ds(r0, DB)])

        return carry

    lax.fori_loop(0, KMAX_D, write_chunk, 0)


@functools.partial(
    pl.kernel,
    out_type=jax.ShapeDtypeStruct((NC, N, HALF), jnp.float32),
    mesh=_mesh,
    scratch_types=[
        pltpu.VMEM_SHARED((N, HALF), jnp.float32),
        pltpu.VMEM((CKB, EB), jnp.int32),
        pltpu.VMEM((CKB, EB), jnp.int32),
    ] + [pltpu.VMEM((EB, HALF), jnp.float32)] * NBUF
      + [pltpu.SemaphoreType.DMA] * (2 * NBUF),
)
def _edge_kernel(src_hbm, dst_hbm, h_hbm, out_hbm, acc, sids, dids, *bufsem):
    c = lax.axis_index("c")
    s = lax.axis_index("s")
    bufs = bufsem[:NBUF]
    gsems = bufsem[NBUF:2 * NBUF]
    ssems = bufsem[2 * NBUF:]
    rows = bufs[0]
    gsem = gsems[0]
    ssem = ssems[0]

    def init_chunk(k, carry):
        idx = s + NS * k

        @pl.when(idx < NCH_E)
        def _():
            r0 = idx * EB
            pltpu.sync_copy(h_hbm.at[c].at[pl.ds(r0, EB)], rows)
            pltpu.sync_copy(rows, acc.at[pl.ds(r0, EB)])

        return carry

    lax.fori_loop(0, KMAX_E, init_chunk, 0)
    plsc.subcore_barrier()

    def wait_s(b):
        pltpu.make_async_copy(bufs[b], acc.at[dids.at[0]], ssems[b]).wait()

    def id_chunk(j, carry):
        pltpu.sync_copy(src_hbm.at[s].at[j], sids)
        pltpu.sync_copy(dst_hbm.at[s].at[j], dids)

        def ring(k, c2):
            gds = []
            for b in range(NBUF):
                @pl.when(k > 0)
                def _(b=b):
                    wait_s(b)

                gds.append(pltpu.async_copy(
                    h_hbm.at[c].at[sids.at[NBUF * k + b]], bufs[b], gsems[b]))
            for b in range(NBUF):
                gds[b].wait()
                pltpu.async_copy(bufs[b], acc.at[dids.at[NBUF * k + b]],
                                 ssems[b], add=True)
            return c2

        lax.fori_loop(0, CKB // NBUF, ring, 0)

        wait_s(0)
        g = pltpu.async_copy(h_hbm.at[c].at[sids.at[CKB - 1]], rows, gsem)
        g.wait()
        pltpu.async_copy(rows, acc.at[dids.at[CKB - 1]], ssem, add=True)
        wait_s(0)
        for b in range(1, NBUF):
            wait_s(b)
        return carry

    lax.fori_loop(0, NCK, id_chunk, 0)
    plsc.subcore_barrier()

    def write_chunk(k, carry):
        idx = s + NS * k

        @pl.when(idx < NCH_E)
        def _():
            r0 = idx * EB
            pltpu.sync_copy(acc.at[pl.ds(r0, EB)], rows)
            pltpu.sync_copy(rows, out_hbm.at[c].at[pl.ds(r0, EB)])

        return carry

    lax.fori_loop(0, KMAX_E, write_chunk, 0)


R = 1000
GRID = N // R


_KS0 = 0
_KS1 = 43
_KS2 = _KS0 ^ _KS1 ^ 0x1BD11BDA
_ROT = ((13, 15, 26, 6), (17, 29, 16, 24))
_KSCHED = ((_KS1, _KS2), (_KS2, _KS0), (_KS0, _KS1), (_KS1, _KS2),
           (_KS2, _KS0))
_U_LO = float(np.nextafter(np.float32(-1.0), np.float32(0.0)))


def _threefry_noise_block(i):
    u32 = jnp.uint32
    pflat = (lax.broadcasted_iota(jnp.int32, (R, LATENT), 0) * LATENT
             + lax.broadcasted_iota(jnp.int32, (R, LATENT), 1)
             + i * (R * LATENT)).astype(u32)
    x0 = jnp.full((R, LATENT), _KS0, u32)
    x1 = pflat + u32(_KS1)
    for g in range(5):
        for d in _ROT[g % 2]:
            x0 = x0 + x1
            x1 = (x1 << u32(d)) | (x1 >> u32(32 - d))
            x1 = x0 ^ x1
        x0 = x0 + u32(_KSCHED[g][0])
        x1 = x1 + u32(_KSCHED[g][1] + g + 1)
    bits = x0 ^ x1
    f = lax.bitcast_convert_type((bits >> u32(9)) | u32(0x3F800000),
                                 jnp.float32) - 1.0
    lo = jnp.float32(_U_LO)
    u = jnp.maximum(lo, f * (jnp.float32(1.0) - lo) + lo)
    return jnp.float32(1.4142135623730951) * lax.erf_inv(u)


def _k0_body(nz_out):
    nz_out[...] = _threefry_noise_block(pl.program_id(0))


def _tc_k0():
    return pl.pallas_call(
        _k0_body,
        grid=(GRID,),
        in_specs=[],
        out_specs=pl.BlockSpec((R, LATENT), lambda i: (i, 0)),
        out_shape=jax.ShapeDtypeStruct((N, LATENT), jnp.float32),
    )()


def _k1_body(scal, z, nz, hist, wt1, bt1, wt2, bt2, wenc, h_out, dinv_out,
             temb_out):
    sa = scal[0, 0]
    sb = scal[0, 1]
    tf = scal[0, 2]

    deg = hist[0, :, :1] + hist[1, :, :1]
    dinv = lax.rsqrt(deg)
    zt = sa * z[...] + sb * nz[...]
    g = jnp.dot(zt, wenc[...], preferred_element_type=jnp.float32)
    h = dinv * g
    h_out[0] = h[:, :HALF]
    h_out[1] = h[:, HALF:]
    dinv_out[...] = jnp.broadcast_to(dinv, (R, 16))

    v = tf * wt1[...] + bt1[...]
    ge = 0.5 * v * (1.0 + lax.erf(v * 0.7071067811865476))
    temb_out[...] = (
        jnp.dot(ge, wt2[...], preferred_element_type=jnp.float32) + bt2[...])


def _tc_k1(scal, z, nz, hist, wt1, bt1, wt2, bt2, wenc):
    return pl.pallas_call(
        _k1_body,
        grid=(GRID,),
        in_specs=[
            pl.BlockSpec(memory_space=pltpu.SMEM),
            pl.BlockSpec((R, LATENT), lambda i: (i, 0)),
            pl.BlockSpec((R, LATENT), lambda i: (i, 0)),
            pl.BlockSpec((NC, R, 16), lambda i: (0, i, 0)),
            pl.BlockSpec((1, LATENT), lambda i: (0, 0)),
            pl.BlockSpec((1, LATENT), lambda i: (0, 0)),
            pl.BlockSpec((LATENT, LATENT), lambda i: (0, 0)),
            pl.BlockSpec((1, LATENT), lambda i: (0, 0)),
            pl.BlockSpec((LATENT, LATENT), lambda i: (0, 0)),
        ],
        out_specs=[
            pl.BlockSpec((NC, R, HALF), lambda i: (0, i, 0)),
            pl.BlockSpec((R, 16), lambda i: (i, 0)),
            pl.BlockSpec((1, LATENT), lambda i: (0, 0)),
        ],
        out_shape=[
            jax.ShapeDtypeStruct((NC, N, HALF), jnp.float32),
            jax.ShapeDtypeStruct((N, 16), jnp.float32),
            jax.ShapeDtypeStruct((1, LATENT), jnp.float32),
        ],
    )(scal, z, nz, hist, wt1, bt1, wt2, bt2, wenc)


def _k2_body(acc, dinv16, temb, benc, wdec, h_out):
    dinv = dinv16[:, :1]
    o1 = dinv * jnp.concatenate([acc[0], acc[1]], axis=1) + benc[...] + temb[...]
    h1 = jnp.where(o1 > 0.0, o1, jnp.exp(jnp.minimum(o1, 0.0)) - 1.0)
    g = jnp.dot(h1, wdec[...], preferred_element_type=jnp.float32)
    h = dinv * g
    h_out[0] = h[:, :HALF]
    h_out[1] = h[:, HALF:]


def _tc_k2(acc, dinv16, temb, benc, wdec):
    return pl.pallas_call(
        _k2_body,
        grid=(GRID,),
        in_specs=[
            pl.BlockSpec((NC, R, HALF), lambda i: (0, i, 0)),
            pl.BlockSpec((R, 16), lambda i: (i, 0)),
            pl.BlockSpec((1, LATENT), lambda i: (0, 0)),
            pl.BlockSpec((1, LATENT), lambda i: (0, 0)),
            pl.BlockSpec((LATENT, LATENT), lambda i: (0, 0)),
        ],
        out_specs=pl.BlockSpec((NC, R, HALF), lambda i: (0, i, 0)),
        out_shape=jax.ShapeDtypeStruct((NC, N, HALF), jnp.float32),
    )(acc, dinv16, temb, benc, wdec)


def _k3_body(acc, dinv16, bdec, nz, out):
    i = pl.program_id(0)

    @pl.when(i == 0)
    def _():
        out[...] = jnp.zeros((1, HALF), jnp.float32)

    dinv = dinv16[:, :1]
    pred = dinv * jnp.concatenate([acc[0], acc[1]], axis=1) + bdec[...]
    d = pred - nz[...]
    out[...] += jnp.full((1, HALF), jnp.sum(d * d) * (1.0 / (N * LATENT)))


def _tc_k3(acc, dinv16, bdec, nz):
    return pl.pallas_call(
        _k3_body,
        grid=(GRID,),
        in_specs=[
            pl.BlockSpec((NC, R, HALF), lambda i: (0, i, 0)),
            pl.BlockSpec((R, 16), lambda i: (i, 0)),
            pl.BlockSpec((1, LATENT), lambda i: (0, 0)),
            pl.BlockSpec((R, LATENT), lambda i: (i, 0)),
        ],
        out_specs=pl.BlockSpec((1, HALF), lambda i: (0, 0)),
        out_shape=jax.ShapeDtypeStruct((1, HALF), jnp.float32),
    )(acc, dinv16, bdec, nz)


@jax.jit
def kernel(z, edge_index, Wt1, bt1, Wt2, bt2, Wenc, benc, Wdec, bdec):
    beta = jnp.linspace(1e-4, 0.02, T).astype(jnp.float32)
    alpha_bar = jnp.cumprod(1.0 - beta)
    t = jax.random.randint(jax.random.key(42), (1,), 0, T)
    a_bar_t = alpha_bar[t]
    sa = jnp.sqrt(a_bar_t)[0]
    sb = jnp.sqrt(1.0 - a_bar_t)[0]
    tf = t.astype(jnp.float32)[0]
    scal = jnp.stack([sa, sb, tf]).reshape(1, 3)

    src = edge_index[0].reshape(NS, NCK, CKB, EB)
    dst = edge_index[1].reshape(NS, NCK, CKB, EB)
    dst_deg = edge_index[1].reshape(NC * NS, DNCK, DCKB, DB)

    seed = jnp.concatenate(
        [jnp.ones((1, N, 16), jnp.float32), jnp.zeros((1, N, 16), jnp.float32)])
    ones_rows = jnp.ones((DB, 16), jnp.float32)

    bt1r = bt1.reshape(1, LATENT)
    bt2r = bt2.reshape(1, LATENT)
    bencr = benc.reshape(1, LATENT)
    bdecr = bdec.reshape(1, LATENT)

    hist = _deg_kernel(dst_deg, seed, ones_rows)
    noise = _tc_k0()
    h1p, dinv16, temb = _tc_k1(scal, z, noise, hist, Wt1, bt1r, Wt2, bt2r,
                               Wenc)
    acc1 = _edge_kernel(src, dst, h1p)
    h2p = _tc_k2(acc1, dinv16, temb, bencr, Wdec)
    acc2 = _edge_kernel(src, dst, h2p)
    loss = _tc_k3(acc2, dinv16, bdecr, noise)
    return loss[0, 0]


# speedup vs baseline: 1.0790x; 1.0790x over previous
import functools

import jax
import jax.numpy as jnp
import numpy as np
from jax import lax
from jax.experimental import pallas as pl
from jax.experimental.pallas import tpu as pltpu
from jax.experimental.pallas import tpu_sc as plsc

N = 10000
E = 160000
LATENT = 256
T = 1000
HALF = 128

NC = 2
NS = 16

EB = 80
CKB = 25
NCK = 5
NCH_E = N // EB
KMAX_E = (NCH_E + NS - 1) // NS

DB = 40
DCKB = 25
DNCK = 5
NCH_D = N // DB
KMAX_D = (NCH_D + NS - 1) // NS

_mesh = plsc.VectorSubcoreMesh(core_axis_name="c", subcore_axis_name="s")


@functools.partial(
    pl.kernel,
    out_type=jax.ShapeDtypeStruct((NC, N, 16), jnp.float32),
    mesh=_mesh,
    scratch_types=[
        pltpu.VMEM_SHARED((N, 16), jnp.float32),
        pltpu.VMEM((DCKB, DB), jnp.int32),
        pltpu.VMEM((DB, 16), jnp.float32),
        pltpu.VMEM((DB, 16), jnp.float32),
        pltpu.SemaphoreType.DMA,
    ],
)
def _deg_kernel(dst_hbm, seed_hbm, ones_hbm, out_hbm, hist, ids, ones, tmp,
                dsem):
    c = lax.axis_index("c")
    s = lax.axis_index("s")
    wid = c * NS + s

    pltpu.sync_copy(ones_hbm, ones)

    def init_chunk(k, carry):
        idx = s + NS * k

        @pl.when(idx < NCH_D)
        def _():
            r0 = idx * DB
            pltpu.sync_copy(seed_hbm.at[c].at[pl.ds(r0, DB)], tmp)
            pltpu.sync_copy(tmp, hist.at[pl.ds(r0, DB)])

        return carry

    lax.fori_loop(0, KMAX_D, init_chunk, 0)
    plsc.subcore_barrier()

    def id_chunk(j, carry):
        pltpu.sync_copy(dst_hbm.at[wid].at[j], ids)

        def fire(i, c2):
            pltpu.async_copy(ones, hist.at[ids.at[i]], dsem, add=True)
            return c2

        lax.fori_loop(0, DCKB, fire, 0)

        def drain(i, c2):
            pltpu.make_async_copy(ones, hist.at[ids.at[0]], dsem).wait()
            return c2

        lax.fori_loop(0, DCKB, drain, 0)
        return carry

    lax.fori_loop(0, DNCK, id_chunk, 0)
    plsc.subcore_barrier()

    def write_chunk(k, carry):
        idx = s + NS * k

        @pl.when(idx < NCH_D)
        def _():
            r0 = idx * DB
            pltpu.sync_copy(hist.at[pl.ds(r0, DB)], tmp)
            pltpu.sync_copy(tmp, out_hbm.at[c].at[pl.ds(r0, DB)])

        return carry

    lax.fori_loop(0, KMAX_D, write_chunk, 0)


@functools.partial(
    pl.kernel,
    out_type=jax.ShapeDtypeStruct((NC, N, HALF), jnp.float32),
    mesh=_mesh,
    scratch_types=[
        pltpu.VMEM_SHARED((N, HALF), jnp.float32),
        pltpu.VMEM((CKB, EB), jnp.int32),
        pltpu.VMEM((CKB, EB), jnp.int32),
        pltpu.VMEM((EB, HALF), jnp.float32),
        pltpu.VMEM((EB, HALF), jnp.float32),
        pltpu.VMEM((EB, HALF), jnp.float32),
        pltpu.VMEM((EB, HALF), jnp.float32),
        pltpu.SemaphoreType.DMA,
        pltpu.SemaphoreType.DMA,
        pltpu.SemaphoreType.DMA,
        pltpu.SemaphoreType.DMA,
        pltpu.SemaphoreType.DMA,
        pltpu.SemaphoreType.DMA,
        pltpu.SemaphoreType.DMA,
        pltpu.SemaphoreType.DMA,
    ],
)
def _edge_kernel(src_hbm, dst_hbm, h_hbm, out_hbm, acc, sids, dids, rows,
                 rows1, rows2, rows3, gsem, gsem1, gsem2, gsem3, ssem, ssem1,
                 ssem2, ssem3):
    c = lax.axis_index("c")
    s = lax.axis_index("s")
    bufs = (rows, rows1, rows2, rows3)
    ssems = (ssem, ssem1, ssem2, ssem3)
    gsems = (gsem, gsem1, gsem2, gsem3)

    def init_chunk(k, carry):
        idx = s + NS * k

        @pl.when(idx < NCH_E)
        def _():
            r0 = idx * EB
            pltpu.sync_copy(h_hbm.at[c].at[pl.ds(r0, EB)], rows)
            pltpu.sync_copy(rows, acc.at[pl.ds(r0, EB)])

        return carry

    lax.fori_loop(0, KMAX_E, init_chunk, 0)
    plsc.subcore_barrier()

    def wait_s(b):
        pltpu.make_async_copy(bufs[b], acc.at[dids.at[0]], ssems[b]).wait()

    def id_chunk(j, carry):
        pltpu.sync_copy(src_hbm.at[s].at[j], sids)
        pltpu.sync_copy(dst_hbm.at[s].at[j], dids)

        def quad(k, c2):
            gds = []
            for b in range(4):
                @pl.when(k > 0)
                def _(b=b):
                    wait_s(b)

                gds.append(pltpu.async_copy(
                    h_hbm.at[c].at[sids.at[4 * k + b]], bufs[b], gsems[b]))
            for b in range(4):
                gds[b].wait()
                pltpu.async_copy(bufs[b], acc.at[dids.at[4 * k + b]],
                                 ssems[b], add=True)
            return c2

        lax.fori_loop(0, CKB // 4, quad, 0)

        wait_s(0)
        g = pltpu.async_copy(h_hbm.at[c].at[sids.at[CKB - 1]], rows, gsem)
        g.wait()
        pltpu.async_copy(rows, acc.at[dids.at[CKB - 1]], ssem, add=True)
        wait_s(0)
        wait_s(1)
        wait_s(2)
        wait_s(3)
        return carry

    lax.fori_loop(0, NCK, id_chunk, 0)
    plsc.subcore_barrier()

    def write_chunk(k, carry):
        idx = s + NS * k

        @pl.when(idx < NCH_E)
        def _():
            r0 = idx * EB
            pltpu.sync_copy(acc.at[pl.ds(r0, EB)], rows)
            pltpu.sync_copy(rows, out_hbm.at[c].at[pl.ds(r0, EB)])

        return carry

    lax.fori_loop(0, KMAX_E, write_chunk, 0)


R = 1000
GRID = N // R


_KS0 = 0
_KS1 = 43
_KS2 = _KS0 ^ _KS1 ^ 0x1BD11BDA
_ROT = ((13, 15, 26, 6), (17, 29, 16, 24))
_KSCHED = ((_KS1, _KS2), (_KS2, _KS0), (_KS0, _KS1), (_KS1, _KS2),
           (_KS2, _KS0))
_U_LO = float(np.nextafter(np.float32(-1.0), np.float32(0.0)))


def _threefry_noise_block(i):
    u32 = jnp.uint32
    pflat = (lax.broadcasted_iota(jnp.int32, (R, LATENT), 0) * LATENT
             + lax.broadcasted_iota(jnp.int32, (R, LATENT), 1)
             + i * (R * LATENT)).astype(u32)
    x0 = jnp.full((R, LATENT), _KS0, u32)
    x1 = pflat + u32(_KS1)
    for g in range(5):
        for d in _ROT[g % 2]:
            x0 = x0 + x1
            x1 = (x1 << u32(d)) | (x1 >> u32(32 - d))
            x1 = x0 ^ x1
        x0 = x0 + u32(_KSCHED[g][0])
        x1 = x1 + u32(_KSCHED[g][1] + g + 1)
    bits = x0 ^ x1
    f = lax.bitcast_convert_type((bits >> u32(9)) | u32(0x3F800000),
                                 jnp.float32) - 1.0
    lo = jnp.float32(_U_LO)
    u = jnp.maximum(lo, f * (jnp.float32(1.0) - lo) + lo)
    return jnp.float32(1.4142135623730951) * lax.erf_inv(u)


def _k0_body(nz_out):
    nz_out[...] = _threefry_noise_block(pl.program_id(0))


def _tc_k0():
    return pl.pallas_call(
        _k0_body,
        grid=(GRID,),
        in_specs=[],
        out_specs=pl.BlockSpec((R, LATENT), lambda i: (i, 0)),
        out_shape=jax.ShapeDtypeStruct((N, LATENT), jnp.float32),
    )()


def _k1_body(scal, z, nz, hist, wt1, bt1, wt2, bt2, wenc, h_out, dinv_out,
             temb_out):
    sa = scal[0, 0]
    sb = scal[0, 1]
    tf = scal[0, 2]

    deg = hist[0, :, :1] + hist[1, :, :1]
    dinv = lax.rsqrt(deg)
    zt = sa * z[...] + sb * nz[...]
    g = jnp.dot(zt, wenc[...], preferred_element_type=jnp.float32)
    h = dinv * g
    h_out[0] = h[:, :HALF]
    h_out[1] = h[:, HALF:]
    dinv_out[...] = jnp.broadcast_to(dinv, (R, 16))

    v = tf * wt1[...] + bt1[...]
    ge = 0.5 * v * (1.0 + lax.erf(v * 0.7071067811865476))
    temb_out[...] = (
        jnp.dot(ge, wt2[...], preferred_element_type=jnp.float32) + bt2[...])


def _tc_k1(scal, z, nz, hist, wt1, bt1, wt2, bt2, wenc):
    return pl.pallas_call(
        _k1_body,
        grid=(GRID,),
        in_specs=[
            pl.BlockSpec(memory_space=pltpu.SMEM),
            pl.BlockSpec((R, LATENT), lambda i: (i, 0)),
            pl.BlockSpec((R, LATENT), lambda i: (i, 0)),
            pl.BlockSpec((NC, R, 16), lambda i: (0, i, 0)),
            pl.BlockSpec((1, LATENT), lambda i: (0, 0)),
            pl.BlockSpec((1, LATENT), lambda i: (0, 0)),
            pl.BlockSpec((LATENT, LATENT), lambda i: (0, 0)),
            pl.BlockSpec((1, LATENT), lambda i: (0, 0)),
            pl.BlockSpec((LATENT, LATENT), lambda i: (0, 0)),
        ],
        out_specs=[
            pl.BlockSpec((NC, R, HALF), lambda i: (0, i, 0)),
            pl.BlockSpec((R, 16), lambda i: (i, 0)),
            pl.BlockSpec((1, LATENT), lambda i: (0, 0)),
        ],
        out_shape=[
            jax.ShapeDtypeStruct((NC, N, HALF), jnp.float32),
            jax.ShapeDtypeStruct((N, 16), jnp.float32),
            jax.ShapeDtypeStruct((1, LATENT), jnp.float32),
        ],
    )(scal, z, nz, hist, wt1, bt1, wt2, bt2, wenc)


def _k2_body(acc, dinv16, temb, benc, wdec, h_out):
    dinv = dinv16[:, :1]
    o1 = dinv * jnp.concatenate([acc[0], acc[1]], axis=1) + benc[...] + temb[...]
    h1 = jnp.where(o1 > 0.0, o1, jnp.exp(jnp.minimum(o1, 0.0)) - 1.0)
    g = jnp.dot(h1, wdec[...], preferred_element_type=jnp.float32)
    h = dinv * g
    h_out[0] = h[:, :HALF]
    h_out[1] = h[:, HALF:]


def _tc_k2(acc, dinv16, temb, benc, wdec):
    return pl.pallas_call(
        _k2_body,
        grid=(GRID,),
        in_specs=[
            pl.BlockSpec((NC, R, HALF), lambda i: (0, i, 0)),
            pl.BlockSpec((R, 16), lambda i: (i, 0)),
            pl.BlockSpec((1, LATENT), lambda i: (0, 0)),
            pl.BlockSpec((1, LATENT), lambda i: (0, 0)),
            pl.BlockSpec((LATENT, LATENT), lambda i: (0, 0)),
        ],
        out_specs=pl.BlockSpec((NC, R, HALF), lambda i: (0, i, 0)),
        out_shape=jax.ShapeDtypeStruct((NC, N, HALF), jnp.float32),
    )(acc, dinv16, temb, benc, wdec)


def _k3_body(acc, dinv16, bdec, nz, out):
    i = pl.program_id(0)

    @pl.when(i == 0)
    def _():
        out[...] = jnp.zeros((1, HALF), jnp.float32)

    dinv = dinv16[:, :1]
    pred = dinv * jnp.concatenate([acc[0], acc[1]], axis=1) + bdec[...]
    d = pred - nz[...]
    out[...] += jnp.full((1, HALF), jnp.sum(d * d) * (1.0 / (N * LATENT)))


def _tc_k3(acc, dinv16, bdec, nz):
    return pl.pallas_call(
        _k3_body,
        grid=(GRID,),
        in_specs=[
            pl.BlockSpec((NC, R, HALF), lambda i: (0, i, 0)),
            pl.BlockSpec((R, 16), lambda i: (i, 0)),
            pl.BlockSpec((1, LATENT), lambda i: (0, 0)),
            pl.BlockSpec((R, LATENT), lambda i: (i, 0)),
        ],
        out_specs=pl.BlockSpec((1, HALF), lambda i: (0, 0)),
        out_shape=jax.ShapeDtypeStruct((1, HALF), jnp.float32),
    )(acc, dinv16, bdec, nz)


@jax.jit
def kernel(z, edge_index, Wt1, bt1, Wt2, bt2, Wenc, benc, Wdec, bdec):
    beta = jnp.linspace(1e-4, 0.02, T).astype(jnp.float32)
    alpha_bar = jnp.cumprod(1.0 - beta)
    t = jax.random.randint(jax.random.key(42), (1,), 0, T)
    a_bar_t = alpha_bar[t]
    sa = jnp.sqrt(a_bar_t)[0]
    sb = jnp.sqrt(1.0 - a_bar_t)[0]
    tf = t.astype(jnp.float32)[0]
    scal = jnp.stack([sa, sb, tf]).reshape(1, 3)

    src = edge_index[0].reshape(NS, NCK, CKB, EB)
    dst = edge_index[1].reshape(NS, NCK, CKB, EB)
    dst_deg = edge_index[1].reshape(NC * NS, DNCK, DCKB, DB)

    seed = jnp.concatenate(
        [jnp.ones((1, N, 16), jnp.float32), jnp.zeros((1, N, 16), jnp.float32)])
    ones_rows = jnp.ones((DB, 16), jnp.float32)

    bt1r = bt1.reshape(1, LATENT)
    bt2r = bt2.reshape(1, LATENT)
    bencr = benc.reshape(1, LATENT)
    bdecr = bdec.reshape(1, LATENT)

    hist = _deg_kernel(dst_deg, seed, ones_rows)
    noise = _tc_k0()
    h1p, dinv16, temb = _tc_k1(scal, z, noise, hist, Wt1, bt1r, Wt2, bt2r,
                               Wenc)
    acc1 = _edge_kernel(src, dst, h1p)
    h2p = _tc_k2(acc1, dinv16, temb, bencr, Wdec)
    acc2 = _edge_kernel(src, dst, h2p)
    loss = _tc_k3(acc2, dinv16, bdecr, noise)
    return loss[0, 0]
